# Initial kernel scaffold; baseline (speedup 1.0000x reference)
import jax
import jax.numpy as jnp
from jax.experimental import pallas as pl


def kernel(inputs, embedding):
    raise NotImplementedError("write your pallas kernel here")


# speedup vs baseline: 1.0853x; 1.0853x over previous
import functools

import jax
import jax.numpy as jnp
from jax import lax
from jax.experimental import pallas as pl
from jax.experimental.pallas import tpu as pltpu
from jax.experimental.pallas import tpu_sc as plsc

N_EMB = 8192
DIM = 256
TOKENS = 18432
COMMIT = 0.25

TILE_N = 256
N_TILES = TOKENS // TILE_N
K_CHUNK = 1024
N_CHUNKS = N_EMB // K_CHUNK

_LOSS_SCALE = 2.0 * COMMIT / (TOKENS * DIM)


def _vq_argmin_body(z_ref, et_ref, idx_ref, loss_ref):
    i = pl.program_id(0)

    @pl.when(i == 0)
    def _():
        loss_ref[...] = jnp.zeros_like(loss_ref)

    z = z_ref[...]
    z2 = jnp.sum(z * z, axis=1, keepdims=True)
    rmin = None
    ridx = None
    for c in range(N_CHUNKS):
        ec = et_ref[:, c * K_CHUNK:(c + 1) * K_CHUNK]
        e2 = jnp.sum(ec * ec, axis=0, keepdims=True)
        m = lax.dot_general(z, ec, (((1,), (0,)), ((), ())),
                            preferred_element_type=jnp.float32)
        d = (z2 + e2) - 2.0 * m
        cmin = jnp.min(d, axis=1, keepdims=True)
        iot = lax.broadcasted_iota(jnp.int32, d.shape, 1)
        cidx = jnp.min(jnp.where(d == cmin, iot, jnp.int32(2**30)),
                       axis=1, keepdims=True) + c * K_CHUNK
        if c == 0:
            rmin, ridx = cmin, cidx
        else:
            upd = cmin < rmin
            rmin = jnp.where(upd, cmin, rmin)
            ridx = jnp.where(upd, cidx, ridx)
    idx_ref[...] = ridx
    loss_ref[...] += jnp.sum(rmin, keepdims=True)

    @pl.when(i == N_TILES - 1)
    def _():
        loss_ref[...] = loss_ref[...] * _LOSS_SCALE


def _argmin_call(z_flat, et):
    return pl.pallas_call(
        _vq_argmin_body,
        grid=(N_TILES,),
        in_specs=[
            pl.BlockSpec((TILE_N, DIM), lambda i: (i, 0)),
            pl.BlockSpec((DIM, N_EMB), lambda i: (0, 0)),
        ],
        out_specs=[
            pl.BlockSpec((TILE_N, 1), lambda i: (i, 0)),
            pl.BlockSpec((1, 1), lambda i: (0, 0)),
        ],
        out_shape=[
            jax.ShapeDtypeStruct((TOKENS, 1), jnp.int32),
            jax.ShapeDtypeStruct((1, 1), jnp.float32),
        ],
    )(z_flat, et)


_NC, _NS = 2, 16
_NW = _NC * _NS
_ROWS_PER_W = TOKENS // _NW
_GCHUNK = 96
_GCHUNKS = _ROWS_PER_W // _GCHUNK


def _sc_gather_body(table_hbm, idx_hbm, out_hbm, idx_v, rows_a, rows_b,
                    sem_a, sem_b):
    wid = lax.axis_index("s") * _NC + lax.axis_index("c")
    pltpu.sync_copy(idx_hbm.at[wid], idx_v)
    bufs = (rows_a, rows_b)
    sems = (sem_a, sem_b)
    prev = pltpu.async_copy(table_hbm.at[idx_v.at[0]], bufs[0], sems[0])
    for c in range(_GCHUNKS):
        nxt = None
        if c + 1 < _GCHUNKS:
            nxt = pltpu.async_copy(table_hbm.at[idx_v.at[c + 1]],
                                   bufs[(c + 1) % 2], sems[(c + 1) % 2])
        prev.wait()
        pltpu.sync_copy(bufs[c % 2], out_hbm.at[wid, c])
        prev = nxt


def _sc_gather(embedding, idx_sc):
    mesh = plsc.VectorSubcoreMesh(core_axis_name="c", subcore_axis_name="s")
    return pl.kernel(
        _sc_gather_body,
        mesh=mesh,
        out_type=jax.ShapeDtypeStruct((_NW, _GCHUNKS, _GCHUNK, DIM),
                                      jnp.float32),
        scratch_types=[
            pltpu.VMEM((_GCHUNKS, _GCHUNK), jnp.int32),
            pltpu.VMEM((_GCHUNK, DIM), jnp.float32),
            pltpu.VMEM((_GCHUNK, DIM), jnp.float32),
            pltpu.SemaphoreType.DMA,
            pltpu.SemaphoreType.DMA,
        ],
    )(embedding, idx_sc)


def kernel(inputs, embedding):
    z_flat = inputs.reshape(-1, DIM)
    et = embedding.T
    encoding_indices, loss = _argmin_call(z_flat, et)
    idx_sc = encoding_indices.reshape(_NW, _GCHUNKS, _GCHUNK)
    quantized = _sc_gather(embedding, idx_sc).reshape(inputs.shape)
    return quantized, encoding_indices, loss[0, 0]


# speedup vs baseline: 1.1685x; 1.0767x over previous
import functools

import jax
import jax.numpy as jnp
from jax import lax
from jax.experimental import pallas as pl
from jax.experimental.pallas import tpu as pltpu
from jax.experimental.pallas import tpu_sc as plsc

N_EMB = 8192
DIM = 256
TOKENS = 18432
COMMIT = 0.25

TILE_N = 256
N_TILES = TOKENS // TILE_N
K_CHUNK = 1024
N_CHUNKS = N_EMB // K_CHUNK

_LOSS_SCALE = 2.0 * COMMIT / (TOKENS * DIM)


def _vq_argmin_body(z_ref, e_ref, idx_ref, loss_ref, et_ref, e2_ref):
    i = pl.program_id(0)

    @pl.when(i == 0)
    def _():
        loss_ref[...] = jnp.zeros_like(loss_ref)
        for c in range(N_CHUNKS):
            ec = e_ref[c * K_CHUNK:(c + 1) * K_CHUNK, :]
            ett = lax.transpose(ec, (1, 0))
            et_ref[:, c * K_CHUNK:(c + 1) * K_CHUNK] = ett
            e2_ref[:, c * K_CHUNK:(c + 1) * K_CHUNK] = jnp.sum(
                ett * ett, axis=0, keepdims=True)

    z = z_ref[...]
    z2 = jnp.sum(z * z, axis=1, keepdims=True)
    rmin = None
    ridx = None
    for c in range(N_CHUNKS):
        ec = et_ref[:, c * K_CHUNK:(c + 1) * K_CHUNK]
        e2 = e2_ref[:, c * K_CHUNK:(c + 1) * K_CHUNK]
        m = lax.dot_general(z, ec, (((1,), (0,)), ((), ())),
                            preferred_element_type=jnp.float32)
        d = (z2 + e2) - 2.0 * m
        cmin = jnp.min(d, axis=1, keepdims=True)
        iot = lax.broadcasted_iota(jnp.int32, d.shape, 1)
        cidx = jnp.min(jnp.where(d == cmin, iot, jnp.int32(2**30)),
                       axis=1, keepdims=True) + c * K_CHUNK
        if c == 0:
            rmin, ridx = cmin, cidx
        else:
            upd = cmin < rmin
            rmin = jnp.where(upd, cmin, rmin)
            ridx = jnp.where(upd, cidx, ridx)
    idx_ref[...] = ridx
    loss_ref[...] += jnp.sum(rmin, keepdims=True)

    @pl.when(i == N_TILES - 1)
    def _():
        loss_ref[...] = loss_ref[...] * _LOSS_SCALE


def _argmin_call(z_flat, emb):
    return pl.pallas_call(
        _vq_argmin_body,
        grid=(N_TILES,),
        in_specs=[
            pl.BlockSpec((TILE_N, DIM), lambda i: (i, 0)),
            pl.BlockSpec((N_EMB, DIM), lambda i: (0, 0)),
        ],
        out_specs=[
            pl.BlockSpec((TILE_N, 1), lambda i: (i, 0)),
            pl.BlockSpec((1, 1), lambda i: (0, 0)),
        ],
        out_shape=[
            jax.ShapeDtypeStruct((TOKENS, 1), jnp.int32),
            jax.ShapeDtypeStruct((1, 1), jnp.float32),
        ],
        scratch_shapes=[
            pltpu.VMEM((DIM, N_EMB), jnp.float32),
            pltpu.VMEM((1, N_EMB), jnp.float32),
        ],
    )(z_flat, emb)


_NC, _NS = 2, 16
_NW = _NC * _NS
_ROWS_PER_W = TOKENS // _NW
_GCHUNK = 96
_GCHUNKS = _ROWS_PER_W // _GCHUNK


def _sc_gather_body(table_hbm, idx_hbm, out_hbm, idx_v, rows_a, rows_b,
                    sem_a, sem_b):
    wid = lax.axis_index("s") * _NC + lax.axis_index("c")
    base = wid * _ROWS_PER_W
    pltpu.sync_copy(idx_hbm.at[pl.ds(base, _ROWS_PER_W)], idx_v)
    bufs = (rows_a, rows_b)
    sems = (sem_a, sem_b)
    prev = pltpu.async_copy(table_hbm.at[idx_v.at[pl.ds(0, _GCHUNK)]],
                            bufs[0], sems[0])
    for c in range(_GCHUNKS):
        nxt = None
        if c + 1 < _GCHUNKS:
            nxt = pltpu.async_copy(
                table_hbm.at[idx_v.at[pl.ds((c + 1) * _GCHUNK, _GCHUNK)]],
                bufs[(c + 1) % 2], sems[(c + 1) % 2])
        prev.wait()
        pltpu.sync_copy(bufs[c % 2], out_hbm.at[pl.ds(base + c * _GCHUNK,
                                                      _GCHUNK)])
        prev = nxt


def _sc_gather(embedding, idx_flat):
    mesh = plsc.VectorSubcoreMesh(core_axis_name="c", subcore_axis_name="s")
    return pl.kernel(
        _sc_gather_body,
        mesh=mesh,
        out_type=jax.ShapeDtypeStruct((TOKENS, DIM), jnp.float32),
        scratch_types=[
            pltpu.VMEM((_ROWS_PER_W,), jnp.int32),
            pltpu.VMEM((_GCHUNK, DIM), jnp.float32),
            pltpu.VMEM((_GCHUNK, DIM), jnp.float32),
            pltpu.SemaphoreType.DMA,
            pltpu.SemaphoreType.DMA,
        ],
    )(embedding, idx_flat)


def kernel(inputs, embedding):
    z_flat = inputs.reshape(-1, DIM)
    encoding_indices, loss = _argmin_call(z_flat, embedding)
    idx_flat = encoding_indices.reshape(TOKENS)
    quantized = _sc_gather(embedding, idx_flat).reshape(inputs.shape)
    return quantized, encoding_indices, loss[0, 0]


# speedup vs baseline: 1.1902x; 1.0186x over previous
import functools

import jax
import jax.numpy as jnp
from jax import lax
from jax.experimental import pallas as pl
from jax.experimental.pallas import tpu as pltpu
from jax.experimental.pallas import tpu_sc as plsc

N_EMB = 8192
DIM = 256
TOKENS = 18432
COMMIT = 0.25

TILE_N = 512
N_TILES = TOKENS // TILE_N
K_CHUNK = 1024
N_CHUNKS = N_EMB // K_CHUNK

_LOSS_SCALE = 2.0 * COMMIT / (TOKENS * DIM)


def _vq_argmin_body(z_ref, e_ref, idx_ref, loss_ref, et_ref, e2_ref):
    i = pl.program_id(0)

    @pl.when(i == 0)
    def _():
        loss_ref[...] = jnp.zeros_like(loss_ref)
        for c in range(N_CHUNKS):
            ec = e_ref[c * K_CHUNK:(c + 1) * K_CHUNK, :]
            ett = lax.transpose(ec, (1, 0))
            et_ref[:, c * K_CHUNK:(c + 1) * K_CHUNK] = ett
            e2_ref[:, c * K_CHUNK:(c + 1) * K_CHUNK] = jnp.sum(
                ett * ett, axis=0, keepdims=True)

    z = z_ref[...]
    z2 = jnp.sum(z * z, axis=1, keepdims=True)
    rmin = None
    ridx = None
    for c in range(N_CHUNKS):
        ec = et_ref[:, c * K_CHUNK:(c + 1) * K_CHUNK]
        e2 = e2_ref[:, c * K_CHUNK:(c + 1) * K_CHUNK]
        m = lax.dot_general(z, ec, (((1,), (0,)), ((), ())),
                            preferred_element_type=jnp.float32)
        d = (z2 + e2) - 2.0 * m
        cmin = jnp.min(d, axis=1, keepdims=True)
        iot = lax.broadcasted_iota(jnp.int32, d.shape, 1)
        cidx = jnp.min(jnp.where(d == cmin, iot, jnp.int32(2**30)),
                       axis=1, keepdims=True) + c * K_CHUNK
        if c == 0:
            rmin, ridx = cmin, cidx
        else:
            upd = cmin < rmin
            rmin = jnp.where(upd, cmin, rmin)
            ridx = jnp.where(upd, cidx, ridx)
    idx_ref[...] = ridx
    loss_ref[...] += jnp.sum(rmin, keepdims=True)

    @pl.when(i == N_TILES - 1)
    def _():
        loss_ref[...] = loss_ref[...] * _LOSS_SCALE


def _argmin_call(z_flat, emb):
    return pl.pallas_call(
        _vq_argmin_body,
        grid=(N_TILES,),
        in_specs=[
            pl.BlockSpec((TILE_N, DIM), lambda i: (i, 0)),
            pl.BlockSpec((N_EMB, DIM), lambda i: (0, 0)),
        ],
        out_specs=[
            pl.BlockSpec((TILE_N, 1), lambda i: (i, 0)),
            pl.BlockSpec((1, 1), lambda i: (0, 0)),
        ],
        out_shape=[
            jax.ShapeDtypeStruct((TOKENS, 1), jnp.int32),
            jax.ShapeDtypeStruct((1, 1), jnp.float32),
        ],
        scratch_shapes=[
            pltpu.VMEM((DIM, N_EMB), jnp.float32),
            pltpu.VMEM((1, N_EMB), jnp.float32),
        ],
    )(z_flat, emb)


_NC, _NS = 2, 16
_NW = _NC * _NS
_ROWS_PER_W = TOKENS // _NW
_GCHUNK = 96
_GCHUNKS = _ROWS_PER_W // _GCHUNK


def _sc_gather_body(table_hbm, idx_hbm, out_hbm, idx_v, rows_a, rows_b,
                    sem_a, sem_b):
    wid = lax.axis_index("s") * _NC + lax.axis_index("c")
    base = pl.multiple_of(wid * _ROWS_PER_W, 8)
    pltpu.sync_copy(idx_hbm.at[pl.ds(base, _ROWS_PER_W)], idx_v)
    bufs = (rows_a, rows_b)
    sems = (sem_a, sem_b)
    prev = pltpu.async_copy(table_hbm.at[idx_v.at[pl.ds(0, _GCHUNK)]],
                            bufs[0], sems[0])
    for c in range(_GCHUNKS):
        nxt = None
        if c + 1 < _GCHUNKS:
            nxt = pltpu.async_copy(
                table_hbm.at[idx_v.at[pl.ds((c + 1) * _GCHUNK, _GCHUNK)]],
                bufs[(c + 1) % 2], sems[(c + 1) % 2])
        prev.wait()
        pltpu.sync_copy(bufs[c % 2], out_hbm.at[pl.ds(base + c * _GCHUNK,
                                                      _GCHUNK)])
        prev = nxt


def _sc_gather(embedding, idx_flat):
    mesh = plsc.VectorSubcoreMesh(core_axis_name="c", subcore_axis_name="s")
    return pl.kernel(
        _sc_gather_body,
        mesh=mesh,
        out_type=jax.ShapeDtypeStruct((TOKENS, DIM), jnp.float32),
        scratch_types=[
            pltpu.VMEM((_ROWS_PER_W,), jnp.int32),
            pltpu.VMEM((_GCHUNK, DIM), jnp.float32),
            pltpu.VMEM((_GCHUNK, DIM), jnp.float32),
            pltpu.SemaphoreType.DMA,
            pltpu.SemaphoreType.DMA,
        ],
    )(embedding, idx_flat)


def kernel(inputs, embedding):
    z_flat = inputs.reshape(-1, DIM)
    encoding_indices, loss = _argmin_call(z_flat, embedding)
    idx_flat = encoding_indices.reshape(TOKENS)
    quantized = _sc_gather(embedding, idx_flat).reshape(inputs.shape)
    return quantized, encoding_indices, loss[0, 0]


# speedup vs baseline: 1.3141x; 1.1041x over previous
import functools

import jax
import jax.numpy as jnp
from jax import lax
from jax.experimental import pallas as pl
from jax.experimental.pallas import tpu as pltpu
from jax.experimental.pallas import tpu_sc as plsc

N_EMB = 8192
DIM = 256
TOKENS = 18432
COMMIT = 0.25

TILE_N = 512
N_TILES = TOKENS // TILE_N
K_CHUNK = 1024
N_CHUNKS = N_EMB // K_CHUNK

_LOSS_SCALE = 2.0 * COMMIT / (TOKENS * DIM)


def _vq_argmin_body(z_ref, e_ref, idx_ref, loss_ref, et_ref, e2_ref):
    i = pl.program_id(0)

    @pl.when(i == 0)
    def _():
        loss_ref[...] = jnp.zeros_like(loss_ref)
        for c in range(N_CHUNKS):
            ec = e_ref[c * K_CHUNK:(c + 1) * K_CHUNK, :]
            ett = lax.transpose(ec, (1, 0))
            et_ref[:, c * K_CHUNK:(c + 1) * K_CHUNK] = ett
            e2_ref[:, c * K_CHUNK:(c + 1) * K_CHUNK] = jnp.sum(
                ett * ett, axis=0, keepdims=True)

    z = z_ref[...]
    z2 = jnp.sum(z * z, axis=1, keepdims=True)
    iot = lax.broadcasted_iota(jnp.int32, (TILE_N, K_CHUNK), 1).astype(
        jnp.float32)
    rmin = None
    ridx = None
    for c in range(N_CHUNKS):
        ec = et_ref[:, c * K_CHUNK:(c + 1) * K_CHUNK]
        e2 = e2_ref[:, c * K_CHUNK:(c + 1) * K_CHUNK]
        m = lax.dot_general(z, ec, (((1,), (0,)), ((), ())),
                            preferred_element_type=jnp.float32)
        d = (z2 + e2) - 2.0 * m
        cmin = jnp.min(d, axis=1, keepdims=True)
        cidx = jnp.min(jnp.where(d == cmin, iot, jnp.float32(3e38)),
                       axis=1, keepdims=True) + jnp.float32(c * K_CHUNK)
        if c == 0:
            rmin, ridx = cmin, cidx
        else:
            upd = cmin < rmin
            rmin = jnp.where(upd, cmin, rmin)
            ridx = jnp.where(upd, cidx, ridx)
    idx_ref[...] = ridx.astype(jnp.int32)
    loss_ref[...] += jnp.sum(rmin, keepdims=True)

    @pl.when(i == N_TILES - 1)
    def _():
        loss_ref[...] = loss_ref[...] * _LOSS_SCALE


def _argmin_call(z_flat, emb):
    return pl.pallas_call(
        _vq_argmin_body,
        grid=(N_TILES,),
        in_specs=[
            pl.BlockSpec((TILE_N, DIM), lambda i: (i, 0)),
            pl.BlockSpec((N_EMB, DIM), lambda i: (0, 0)),
        ],
        out_specs=[
            pl.BlockSpec((TILE_N, 1), lambda i: (i, 0)),
            pl.BlockSpec((1, 1), lambda i: (0, 0)),
        ],
        out_shape=[
            jax.ShapeDtypeStruct((TOKENS, 1), jnp.int32),
            jax.ShapeDtypeStruct((1, 1), jnp.float32),
        ],
        scratch_shapes=[
            pltpu.VMEM((DIM, N_EMB), jnp.float32),
            pltpu.VMEM((1, N_EMB), jnp.float32),
        ],
    )(z_flat, emb)


_NC, _NS = 2, 16
_NW = _NC * _NS
_ROWS_PER_W = TOKENS // _NW
_GCHUNK = 96
_GCHUNKS = _ROWS_PER_W // _GCHUNK


def _sc_gather_body(table_hbm, idx_hbm, out_hbm, idx_v, rows_a, rows_b,
                    sem_a, sem_b):
    wid = lax.axis_index("s") * _NC + lax.axis_index("c")
    base = pl.multiple_of(wid * _ROWS_PER_W, 8)
    pltpu.sync_copy(idx_hbm.at[pl.ds(base, _ROWS_PER_W)], idx_v)
    bufs = (rows_a, rows_b)
    sems = (sem_a, sem_b)
    prev = pltpu.async_copy(table_hbm.at[idx_v.at[pl.ds(0, _GCHUNK)]],
                            bufs[0], sems[0])
    for c in range(_GCHUNKS):
        nxt = None
        if c + 1 < _GCHUNKS:
            nxt = pltpu.async_copy(
                table_hbm.at[idx_v.at[pl.ds((c + 1) * _GCHUNK, _GCHUNK)]],
                bufs[(c + 1) % 2], sems[(c + 1) % 2])
        prev.wait()
        pltpu.sync_copy(bufs[c % 2], out_hbm.at[pl.ds(base + c * _GCHUNK,
                                                      _GCHUNK)])
        prev = nxt


def _sc_gather(embedding, idx_flat):
    mesh = plsc.VectorSubcoreMesh(core_axis_name="c", subcore_axis_name="s")
    return pl.kernel(
        _sc_gather_body,
        mesh=mesh,
        out_type=jax.ShapeDtypeStruct((TOKENS, DIM), jnp.float32),
        scratch_types=[
            pltpu.VMEM((_ROWS_PER_W,), jnp.int32),
            pltpu.VMEM((_GCHUNK, DIM), jnp.float32),
            pltpu.VMEM((_GCHUNK, DIM), jnp.float32),
            pltpu.SemaphoreType.DMA,
            pltpu.SemaphoreType.DMA,
        ],
    )(embedding, idx_flat)


def kernel(inputs, embedding):
    z_flat = inputs.reshape(-1, DIM)
    encoding_indices, loss = _argmin_call(z_flat, embedding)
    idx_flat = encoding_indices.reshape(TOKENS)
    quantized = _sc_gather(embedding, idx_flat).reshape(inputs.shape)
    return quantized, encoding_indices, loss[0, 0]


# speedup vs baseline: 1.3473x; 1.0253x over previous
import functools

import jax
import jax.numpy as jnp
from jax import lax
from jax.experimental import pallas as pl
from jax.experimental.pallas import tpu as pltpu
from jax.experimental.pallas import tpu_sc as plsc

N_EMB = 8192
DIM = 256
TOKENS = 18432
COMMIT = 0.25

TILE_N = 512
N_TILES = TOKENS // TILE_N
K_CHUNK = 1024
N_CHUNKS = N_EMB // K_CHUNK

_LOSS_SCALE = 2.0 * COMMIT / (TOKENS * DIM)


def _vq_argmin_body(z_ref, e_ref, idx_ref, loss_ref, et_ref, e2_ref):
    i = pl.program_id(0)

    @pl.when(i == 0)
    def _():
        loss_ref[...] = jnp.zeros_like(loss_ref)
        for c in range(N_CHUNKS):
            ec = e_ref[c * K_CHUNK:(c + 1) * K_CHUNK, :]
            ett = lax.transpose(ec, (1, 0))
            et_ref[:, c * K_CHUNK:(c + 1) * K_CHUNK] = ett
            e2_ref[:, c * K_CHUNK:(c + 1) * K_CHUNK] = jnp.sum(
                ett * ett, axis=0, keepdims=True)

    z = z_ref[...]
    z2 = jnp.sum(z * z, axis=1, keepdims=True)
    zn2 = z * jnp.float32(-2.0)
    iot = lax.broadcasted_iota(jnp.int32, (TILE_N, K_CHUNK), 1).astype(
        jnp.float32)
    rmin = None
    ridx = None
    for c in range(N_CHUNKS):
        ec = et_ref[:, c * K_CHUNK:(c + 1) * K_CHUNK]
        e2 = e2_ref[:, c * K_CHUNK:(c + 1) * K_CHUNK]
        m = lax.dot_general(zn2, ec, (((1,), (0,)), ((), ())),
                            preferred_element_type=jnp.float32)
        d = (z2 + e2) + m
        cmin = jnp.min(d, axis=1, keepdims=True)
        cidx = jnp.min(jnp.where(d == cmin, iot, jnp.float32(3e38)),
                       axis=1, keepdims=True) + jnp.float32(c * K_CHUNK)
        if c == 0:
            rmin, ridx = cmin, cidx
        else:
            upd = cmin < rmin
            rmin = jnp.where(upd, cmin, rmin)
            ridx = jnp.where(upd, cidx, ridx)
    idx_ref[...] = ridx.astype(jnp.int32)
    loss_ref[...] += jnp.sum(rmin, keepdims=True)

    @pl.when(i == N_TILES - 1)
    def _():
        loss_ref[...] = loss_ref[...] * _LOSS_SCALE


def _argmin_call(z_flat, emb):
    return pl.pallas_call(
        _vq_argmin_body,
        grid=(N_TILES,),
        in_specs=[
            pl.BlockSpec((TILE_N, DIM), lambda i: (i, 0)),
            pl.BlockSpec((N_EMB, DIM), lambda i: (0, 0)),
        ],
        out_specs=[
            pl.BlockSpec((TILE_N, 1), lambda i: (i, 0)),
            pl.BlockSpec((1, 1), lambda i: (0, 0)),
        ],
        out_shape=[
            jax.ShapeDtypeStruct((TOKENS, 1), jnp.int32),
            jax.ShapeDtypeStruct((1, 1), jnp.float32),
        ],
        scratch_shapes=[
            pltpu.VMEM((DIM, N_EMB), jnp.float32),
            pltpu.VMEM((1, N_EMB), jnp.float32),
        ],
    )(z_flat, emb)


_NC, _NS = 2, 16
_NW = _NC * _NS
_ROWS_PER_W = TOKENS // _NW
_GCHUNK = 96
_GCHUNKS = _ROWS_PER_W // _GCHUNK


def _sc_gather_body(table_hbm, idx_hbm, out_hbm, idx_v, rows_a, rows_b,
                    sem_a, sem_b):
    wid = lax.axis_index("s") * _NC + lax.axis_index("c")
    base = pl.multiple_of(wid * _ROWS_PER_W, 8)
    pltpu.sync_copy(idx_hbm.at[pl.ds(base, _ROWS_PER_W)], idx_v)
    bufs = (rows_a, rows_b)
    sems = (sem_a, sem_b)
    prev = pltpu.async_copy(table_hbm.at[idx_v.at[pl.ds(0, _GCHUNK)]],
                            bufs[0], sems[0])
    for c in range(_GCHUNKS):
        nxt = None
        if c + 1 < _GCHUNKS:
            nxt = pltpu.async_copy(
                table_hbm.at[idx_v.at[pl.ds((c + 1) * _GCHUNK, _GCHUNK)]],
                bufs[(c + 1) % 2], sems[(c + 1) % 2])
        prev.wait()
        pltpu.sync_copy(bufs[c % 2], out_hbm.at[pl.ds(base + c * _GCHUNK,
                                                      _GCHUNK)])
        prev = nxt


def _sc_gather(embedding, idx_flat):
    mesh = plsc.VectorSubcoreMesh(core_axis_name="c", subcore_axis_name="s")
    return pl.kernel(
        _sc_gather_body,
        mesh=mesh,
        out_type=jax.ShapeDtypeStruct((TOKENS, DIM), jnp.float32),
        scratch_types=[
            pltpu.VMEM((_ROWS_PER_W,), jnp.int32),
            pltpu.VMEM((_GCHUNK, DIM), jnp.float32),
            pltpu.VMEM((_GCHUNK, DIM), jnp.float32),
            pltpu.SemaphoreType.DMA,
            pltpu.SemaphoreType.DMA,
        ],
    )(embedding, idx_flat)


def kernel(inputs, embedding):
    z_flat = inputs.reshape(-1, DIM)
    encoding_indices, loss = _argmin_call(z_flat, embedding)
    idx_flat = encoding_indices.reshape(TOKENS)
    quantized = _sc_gather(embedding, idx_flat).reshape(inputs.shape)
    return quantized, encoding_indices, loss[0, 0]


# speedup vs baseline: 1.3524x; 1.0037x over previous
import functools

import jax
import jax.numpy as jnp
from jax import lax
from jax.experimental import pallas as pl
from jax.experimental.pallas import tpu as pltpu
from jax.experimental.pallas import tpu_sc as plsc

N_EMB = 8192
DIM = 256
TOKENS = 18432
COMMIT = 0.25

TILE_N = 576
N_TILES = TOKENS // TILE_N
K_CHUNK = 1024
N_CHUNKS = N_EMB // K_CHUNK

_LOSS_SCALE = 2.0 * COMMIT / (TOKENS * DIM)


def _vq_argmin_body(z_ref, e_ref, idx_ref, loss_ref, et_ref, e2_ref):
    i = pl.program_id(0)

    @pl.when(i == 0)
    def _():
        loss_ref[...] = jnp.zeros_like(loss_ref)
        for c in range(N_CHUNKS):
            ec = e_ref[c * K_CHUNK:(c + 1) * K_CHUNK, :]
            ett = lax.transpose(ec, (1, 0))
            et_ref[:, c * K_CHUNK:(c + 1) * K_CHUNK] = ett
            e2_ref[:, c * K_CHUNK:(c + 1) * K_CHUNK] = jnp.sum(
                ett * ett, axis=0, keepdims=True)

    z = z_ref[...]
    z2 = jnp.sum(z * z, axis=1, keepdims=True)
    zn2 = z * jnp.float32(-2.0)
    iot = lax.broadcasted_iota(jnp.int32, (TILE_N, K_CHUNK), 1).astype(
        jnp.float32)
    rmin = None
    ridx = None
    for c in range(N_CHUNKS):
        ec = et_ref[:, c * K_CHUNK:(c + 1) * K_CHUNK]
        e2 = e2_ref[:, c * K_CHUNK:(c + 1) * K_CHUNK]
        m = lax.dot_general(zn2, ec, (((1,), (0,)), ((), ())),
                            preferred_element_type=jnp.float32)
        d = (z2 + e2) + m
        cmin = jnp.min(d, axis=1, keepdims=True)
        cidx = jnp.min(jnp.where(d == cmin, iot, jnp.float32(3e38)),
                       axis=1, keepdims=True) + jnp.float32(c * K_CHUNK)
        if c == 0:
            rmin, ridx = cmin, cidx
        else:
            upd = cmin < rmin
            rmin = jnp.where(upd, cmin, rmin)
            ridx = jnp.where(upd, cidx, ridx)
    idx_ref[...] = ridx.astype(jnp.int32)
    loss_ref[...] += jnp.sum(rmin, keepdims=True)

    @pl.when(i == N_TILES - 1)
    def _():
        loss_ref[...] = loss_ref[...] * _LOSS_SCALE


def _argmin_call(z_flat, emb):
    return pl.pallas_call(
        _vq_argmin_body,
        grid=(N_TILES,),
        in_specs=[
            pl.BlockSpec((TILE_N, DIM), lambda i: (i, 0)),
            pl.BlockSpec((N_EMB, DIM), lambda i: (0, 0)),
        ],
        out_specs=[
            pl.BlockSpec((TILE_N, 1), lambda i: (i, 0)),
            pl.BlockSpec((1, 1), lambda i: (0, 0)),
        ],
        out_shape=[
            jax.ShapeDtypeStruct((TOKENS, 1), jnp.int32),
            jax.ShapeDtypeStruct((1, 1), jnp.float32),
        ],
        scratch_shapes=[
            pltpu.VMEM((DIM, N_EMB), jnp.float32),
            pltpu.VMEM((1, N_EMB), jnp.float32),
        ],
    )(z_flat, emb)


_NC, _NS = 2, 16
_NW = _NC * _NS
_ROWS_PER_W = TOKENS // _NW
_GCHUNK = 96
_GCHUNKS = _ROWS_PER_W // _GCHUNK


def _sc_gather_body(table_hbm, idx_hbm, out_hbm, idx_v, rows_a, rows_b,
                    sem_a, sem_b):
    wid = lax.axis_index("s") * _NC + lax.axis_index("c")
    base = pl.multiple_of(wid * _ROWS_PER_W, 8)
    pltpu.sync_copy(idx_hbm.at[pl.ds(base, _ROWS_PER_W)], idx_v)
    bufs = (rows_a, rows_b)
    sems = (sem_a, sem_b)
    prev = pltpu.async_copy(table_hbm.at[idx_v.at[pl.ds(0, _GCHUNK)]],
                            bufs[0], sems[0])
    for c in range(_GCHUNKS):
        nxt = None
        if c + 1 < _GCHUNKS:
            nxt = pltpu.async_copy(
                table_hbm.at[idx_v.at[pl.ds((c + 1) * _GCHUNK, _GCHUNK)]],
                bufs[(c + 1) % 2], sems[(c + 1) % 2])
        prev.wait()
        pltpu.sync_copy(bufs[c % 2], out_hbm.at[pl.ds(base + c * _GCHUNK,
                                                      _GCHUNK)])
        prev = nxt


def _sc_gather(embedding, idx_flat):
    mesh = plsc.VectorSubcoreMesh(core_axis_name="c", subcore_axis_name="s")
    return pl.kernel(
        _sc_gather_body,
        mesh=mesh,
        out_type=jax.ShapeDtypeStruct((TOKENS, DIM), jnp.float32),
        scratch_types=[
            pltpu.VMEM((_ROWS_PER_W,), jnp.int32),
            pltpu.VMEM((_GCHUNK, DIM), jnp.float32),
            pltpu.VMEM((_GCHUNK, DIM), jnp.float32),
            pltpu.SemaphoreType.DMA,
            pltpu.SemaphoreType.DMA,
        ],
    )(embedding, idx_flat)


def kernel(inputs, embedding):
    z_flat = inputs.reshape(-1, DIM)
    encoding_indices, loss = _argmin_call(z_flat, embedding)
    idx_flat = encoding_indices.reshape(TOKENS)
    quantized = _sc_gather(embedding, idx_flat).reshape(inputs.shape)
    return quantized, encoding_indices, loss[0, 0]
